# baseline (speedup 1.0000x reference)
import jax
import jax.numpy as jnp
from jax.experimental import pallas as pl
from jax.experimental.pallas import tpu as pltpu

_B = 128


def _net_kernel(x_ref, a1_ref, b1_ref, a2_ref, b2_ref, w1_ref, bf1_ref,
                wc_ref, bc_ref, o_ref, p1_ref):
    f32 = jnp.float32
    bf16 = jnp.bfloat16
    b = x_ref.shape[1]

    p1_ref[0] = jnp.zeros((b, 512), bf16)
    p1_ref[15] = jnp.zeros((b, 512), bf16)
    p1_ref[:, :, 0:32] = jnp.zeros((16, b, 32), bf16)
    p1_ref[:, :, 480:512] = jnp.zeros((16, b, 32), bf16)

    a1 = a1_ref[...]
    b1 = b1_ref[...]

    def conv1_row(r):
        xr = jnp.concatenate([x_ref[r], x_ref[r + 1], x_ref[r + 2]], axis=1)
        return jnp.dot(xr, a1, preferred_element_type=f32)

    for h2 in range(14):
        m = jnp.maximum(conv1_row(2 * h2), conv1_row(2 * h2 + 1))
        m = jnp.maximum(m + b1, 0.0)
        p = jnp.concatenate(
            [jnp.maximum(m[:, 64 * i:64 * i + 32], m[:, 64 * i + 32:64 * i + 64])
             for i in range(14)], axis=1)
        p1_ref[h2 + 1, :, 32:480] = p.astype(bf16)

    b2 = b2_ref[...]

    def conv2_row(r):
        o = jnp.dot(p1_ref[r], a2_ref[0], preferred_element_type=f32)
        o = o + jnp.dot(p1_ref[r + 1], a2_ref[1], preferred_element_type=f32)
        o = o + jnp.dot(p1_ref[r + 2], a2_ref[2], preferred_element_type=f32)
        return o

    acc = jnp.zeros((b, 128), f32)
    for h in range(7):
        m = jnp.maximum(conv2_row(2 * h), conv2_row(2 * h + 1))
        m = jnp.maximum(m + b2, 0.0)
        p = jnp.concatenate(
            [jnp.maximum(m[:, 128 * i:128 * i + 64], m[:, 128 * i + 64:128 * i + 128])
             for i in range(7)], axis=1)
        acc = acc + jnp.dot(p, w1_ref[h], preferred_element_type=f32)

    h1 = jnp.maximum(acc + bf1_ref[...], 0.0)
    o_ref[...] = jnp.dot(h1, wc_ref[...], preferred_element_type=f32) + bc_ref[...]


def kernel(a1, b1, a2, b2, w_fc1, b_fc1, w_cls, b_cls, x_nchw):
    n = x_nchw.shape[0]
    c = w_cls.shape[1]
    bf16 = jnp.bfloat16
    x = x_nchw[:, 0]
    xt = jnp.pad(x, ((0, 0), (1, 1), (1, 1))).transpose(1, 0, 2).astype(bf16)
    np_ = pl.cdiv(n, _B) * _B
    if np_ != n:
        xt = jnp.pad(xt, ((0, 0), (0, np_ - n), (0, 0)))
    a1c = a1.reshape(90, 896).astype(bf16)
    a2c = a2.astype(bf16)
    w1r = w_fc1.reshape(7, 448, 128)
    out = pl.pallas_call(
        _net_kernel,
        out_shape=jax.ShapeDtypeStruct((np_, c), jnp.float32),
        grid_spec=pltpu.PrefetchScalarGridSpec(
            num_scalar_prefetch=0,
            grid=(np_ // _B,),
            in_specs=[
                pl.BlockSpec((30, _B, 30), lambda i: (0, i, 0)),
                pl.BlockSpec((90, 896), lambda i: (0, 0)),
                pl.BlockSpec((1, 896), lambda i: (0, 0)),
                pl.BlockSpec((3, 512, 896), lambda i: (0, 0, 0)),
                pl.BlockSpec((1, 896), lambda i: (0, 0)),
                pl.BlockSpec((7, 448, 128), lambda i: (0, 0, 0)),
                pl.BlockSpec((1, 128), lambda i: (0, 0)),
                pl.BlockSpec((128, c), lambda i: (0, 0)),
                pl.BlockSpec((1, c), lambda i: (0, 0)),
            ],
            out_specs=pl.BlockSpec((_B, c), lambda i: (i, 0)),
            scratch_shapes=[pltpu.VMEM((16, _B, 512), bf16)],
        ),
        compiler_params=pltpu.CompilerParams(
            dimension_semantics=("parallel",),
            vmem_limit_bytes=48 * 1024 * 1024,
        ),
    )(xt, a1c, b1, a2c, b2, w1r, b_fc1, w_cls, b_cls)
    return out[:n] if np_ != n else out


# speedup vs baseline: 1.0522x; 1.0522x over previous
import jax
import jax.numpy as jnp
from jax.experimental import pallas as pl
from jax.experimental.pallas import tpu as pltpu

_B = 256


def _net_kernel(x_ref, a1_ref, b1_ref, a2_ref, b2_ref, w1_ref, bf1_ref,
                wc_ref, bc_ref, o_ref, p1_ref):
    f32 = jnp.float32
    bf16 = jnp.bfloat16
    b = x_ref.shape[1]

    p1_ref[0] = jnp.zeros((b, 512), bf16)
    p1_ref[15] = jnp.zeros((b, 512), bf16)
    p1_ref[:, :, 0:32] = jnp.zeros((16, b, 32), bf16)
    p1_ref[:, :, 480:512] = jnp.zeros((16, b, 32), bf16)

    a1 = a1_ref[...]
    b1 = b1_ref[...]

    def conv1_row(r):
        xr = jnp.concatenate([x_ref[r], x_ref[r + 1], x_ref[r + 2]], axis=1)
        return jnp.dot(xr, a1, preferred_element_type=f32)

    for h2 in range(14):
        m = jnp.maximum(conv1_row(2 * h2), conv1_row(2 * h2 + 1))
        m = jnp.maximum(m + b1, 0.0).astype(bf16)
        p = jnp.concatenate(
            [jnp.maximum(m[:, 64 * i:64 * i + 32], m[:, 64 * i + 32:64 * i + 64])
             for i in range(14)], axis=1)
        p1_ref[h2 + 1, :, 32:480] = p

    b2 = b2_ref[...]

    def conv2_row(r):
        o = jnp.dot(p1_ref[r], a2_ref[0], preferred_element_type=f32)
        o = o + jnp.dot(p1_ref[r + 1], a2_ref[1], preferred_element_type=f32)
        o = o + jnp.dot(p1_ref[r + 2], a2_ref[2], preferred_element_type=f32)
        return o

    acc = jnp.zeros((b, 128), f32)
    for h in range(7):
        m = jnp.maximum(conv2_row(2 * h), conv2_row(2 * h + 1))
        m = jnp.maximum(m + b2, 0.0)
        p = jnp.concatenate(
            [jnp.maximum(m[:, 128 * i:128 * i + 64], m[:, 128 * i + 64:128 * i + 128])
             for i in range(7)], axis=1)
        acc = acc + jnp.dot(p, w1_ref[h], preferred_element_type=f32)

    h1 = jnp.maximum(acc + bf1_ref[...], 0.0)
    o_ref[...] = jnp.dot(h1, wc_ref[...], preferred_element_type=f32) + bc_ref[...]


def kernel(a1, b1, a2, b2, w_fc1, b_fc1, w_cls, b_cls, x_nchw):
    n = x_nchw.shape[0]
    c = w_cls.shape[1]
    bf16 = jnp.bfloat16
    x = x_nchw[:, 0]
    xt = jnp.pad(x, ((0, 0), (1, 1), (1, 1))).transpose(1, 0, 2).astype(bf16)
    np_ = pl.cdiv(n, _B) * _B
    if np_ != n:
        xt = jnp.pad(xt, ((0, 0), (0, np_ - n), (0, 0)))
    a1c = a1.reshape(90, 896).astype(bf16)
    a2c = a2.astype(bf16)
    w1r = w_fc1.reshape(7, 448, 128)
    out = pl.pallas_call(
        _net_kernel,
        out_shape=jax.ShapeDtypeStruct((np_, c), jnp.float32),
        grid_spec=pltpu.PrefetchScalarGridSpec(
            num_scalar_prefetch=0,
            grid=(np_ // _B,),
            in_specs=[
                pl.BlockSpec((30, _B, 30), lambda i: (0, i, 0)),
                pl.BlockSpec((90, 896), lambda i: (0, 0)),
                pl.BlockSpec((1, 896), lambda i: (0, 0)),
                pl.BlockSpec((3, 512, 896), lambda i: (0, 0, 0)),
                pl.BlockSpec((1, 896), lambda i: (0, 0)),
                pl.BlockSpec((7, 448, 128), lambda i: (0, 0, 0)),
                pl.BlockSpec((1, 128), lambda i: (0, 0)),
                pl.BlockSpec((128, c), lambda i: (0, 0)),
                pl.BlockSpec((1, c), lambda i: (0, 0)),
            ],
            out_specs=pl.BlockSpec((_B, c), lambda i: (i, 0)),
            scratch_shapes=[pltpu.VMEM((16, _B, 512), bf16)],
        ),
        compiler_params=pltpu.CompilerParams(
            dimension_semantics=("parallel",),
            vmem_limit_bytes=48 * 1024 * 1024,
        ),
    )(xt, a1c, b1, a2c, b2, w1r, b_fc1, w_cls, b_cls)
    return out[:n] if np_ != n else out


# speedup vs baseline: 1.1739x; 1.1157x over previous
import jax
import jax.numpy as jnp
from jax.experimental import pallas as pl
from jax.experimental.pallas import tpu as pltpu

_B = 256


def _net_kernel(x_ref, a1_ref, b1_ref, a2_ref, b2_ref, w1_ref, bf1_ref,
                wc_ref, bc_ref, o_ref, p1_ref):
    f32 = jnp.float32
    bf16 = jnp.bfloat16
    b = x_ref.shape[1]

    p1_ref[0] = jnp.zeros((b, 512), bf16)
    p1_ref[15] = jnp.zeros((b, 512), bf16)
    p1_ref[:, :, 0:32] = jnp.zeros((16, b, 32), bf16)
    p1_ref[:, :, 480:512] = jnp.zeros((16, b, 32), bf16)

    a1 = a1_ref[...]
    b1 = b1_ref[...]

    def conv1_row(r):
        xr = jnp.concatenate([x_ref[r], x_ref[r + 1], x_ref[r + 2]], axis=1)
        return jnp.dot(xr, a1, preferred_element_type=f32)

    for h2 in range(14):
        m = jnp.maximum(conv1_row(2 * h2), conv1_row(2 * h2 + 1))
        m = jnp.maximum(m + b1, 0.0).astype(bf16)
        p = jnp.concatenate(
            [jnp.maximum(m[:, 64 * i:64 * i + 32], m[:, 64 * i + 32:64 * i + 64])
             for i in range(14)], axis=1)
        p1_ref[h2 + 1, :, 32:480] = p

    b2 = b2_ref[...]

    def conv2_row(r):
        o = jnp.dot(p1_ref[r], a2_ref[0], preferred_element_type=f32)
        o = o + jnp.dot(p1_ref[r + 1], a2_ref[1], preferred_element_type=f32)
        o = o + jnp.dot(p1_ref[r + 2], a2_ref[2], preferred_element_type=f32)
        return o

    acc = jnp.zeros((b, 128), f32)
    for h in range(7):
        m = jnp.maximum(conv2_row(2 * h), conv2_row(2 * h + 1))
        m = jnp.maximum(m + b2, 0.0).astype(bf16)
        p = jnp.concatenate(
            [jnp.maximum(m[:, 128 * i:128 * i + 64], m[:, 128 * i + 64:128 * i + 128])
             for i in range(7)], axis=1)
        acc = acc + jnp.dot(p, w1_ref[h], preferred_element_type=f32)

    h1 = jnp.maximum(acc + bf1_ref[...], 0.0)
    o_ref[...] = jnp.dot(h1, wc_ref[...], preferred_element_type=f32) + bc_ref[...]


def kernel(a1, b1, a2, b2, w_fc1, b_fc1, w_cls, b_cls, x_nchw):
    n = x_nchw.shape[0]
    c = w_cls.shape[1]
    bf16 = jnp.bfloat16
    x = x_nchw[:, 0]
    xt = jnp.pad(x, ((0, 0), (1, 1), (1, 1))).transpose(1, 0, 2).astype(bf16)
    np_ = pl.cdiv(n, _B) * _B
    if np_ != n:
        xt = jnp.pad(xt, ((0, 0), (0, np_ - n), (0, 0)))
    a1c = a1.reshape(90, 896).astype(bf16)
    a2c = a2.astype(bf16)
    w1r = w_fc1.reshape(7, 448, 128)
    out = pl.pallas_call(
        _net_kernel,
        out_shape=jax.ShapeDtypeStruct((np_, c), jnp.float32),
        grid_spec=pltpu.PrefetchScalarGridSpec(
            num_scalar_prefetch=0,
            grid=(np_ // _B,),
            in_specs=[
                pl.BlockSpec((30, _B, 30), lambda i: (0, i, 0)),
                pl.BlockSpec((90, 896), lambda i: (0, 0)),
                pl.BlockSpec((1, 896), lambda i: (0, 0)),
                pl.BlockSpec((3, 512, 896), lambda i: (0, 0, 0)),
                pl.BlockSpec((1, 896), lambda i: (0, 0)),
                pl.BlockSpec((7, 448, 128), lambda i: (0, 0, 0)),
                pl.BlockSpec((1, 128), lambda i: (0, 0)),
                pl.BlockSpec((128, c), lambda i: (0, 0)),
                pl.BlockSpec((1, c), lambda i: (0, 0)),
            ],
            out_specs=pl.BlockSpec((_B, c), lambda i: (i, 0)),
            scratch_shapes=[pltpu.VMEM((16, _B, 512), bf16)],
        ),
        compiler_params=pltpu.CompilerParams(
            dimension_semantics=("parallel",),
            vmem_limit_bytes=48 * 1024 * 1024,
        ),
    )(xt, a1c, b1, a2c, b2, w1r, b_fc1, w_cls, b_cls)
    return out[:n] if np_ != n else out


# speedup vs baseline: 1.2045x; 1.0261x over previous
import jax
import jax.numpy as jnp
from jax.experimental import pallas as pl
from jax.experimental.pallas import tpu as pltpu

_B = 256


def _net_kernel(x_ref, a1_ref, b1_ref, a2_ref, b2_ref, w1_ref, bf1_ref,
                wc_ref, bc_ref, o_ref, p1_ref):
    f32 = jnp.float32
    bf16 = jnp.bfloat16
    b = x_ref.shape[1]

    p1_ref[0] = jnp.zeros((b, 512), bf16)
    p1_ref[15] = jnp.zeros((b, 512), bf16)
    p1_ref[:, :, 0:32] = jnp.zeros((16, b, 32), bf16)
    p1_ref[:, :, 480:512] = jnp.zeros((16, b, 32), bf16)

    a1 = a1_ref[...]
    b1 = b1_ref[...].astype(bf16)

    def conv1_row(r):
        xr = jnp.concatenate([x_ref[r], x_ref[r + 1], x_ref[r + 2]], axis=1)
        return jnp.dot(xr, a1, preferred_element_type=f32)

    for h2 in range(14):
        m = jnp.maximum(conv1_row(2 * h2), conv1_row(2 * h2 + 1))
        m = jnp.maximum(m.astype(bf16) + b1, 0.0)
        p = jnp.concatenate(
            [jnp.maximum(m[:, 64 * i:64 * i + 32], m[:, 64 * i + 32:64 * i + 64])
             for i in range(14)], axis=1)
        p1_ref[h2 + 1, :, 32:480] = p

    b2 = b2_ref[...].astype(bf16)

    def conv2_row(r):
        o = jnp.dot(p1_ref[r], a2_ref[0], preferred_element_type=f32)
        o = o + jnp.dot(p1_ref[r + 1], a2_ref[1], preferred_element_type=f32)
        o = o + jnp.dot(p1_ref[r + 2], a2_ref[2], preferred_element_type=f32)
        return o

    acc = jnp.zeros((b, 128), f32)
    for h in range(7):
        m = jnp.maximum(conv2_row(2 * h), conv2_row(2 * h + 1))
        m = jnp.maximum(m.astype(bf16) + b2, 0.0)
        p = jnp.concatenate(
            [jnp.maximum(m[:, 128 * i:128 * i + 64], m[:, 128 * i + 64:128 * i + 128])
             for i in range(7)], axis=1)
        acc = acc + jnp.dot(p, w1_ref[h], preferred_element_type=f32)

    h1 = jnp.maximum(acc + bf1_ref[...], 0.0)
    o_ref[...] = jnp.dot(h1, wc_ref[...], preferred_element_type=f32) + bc_ref[...]


def kernel(a1, b1, a2, b2, w_fc1, b_fc1, w_cls, b_cls, x_nchw):
    n = x_nchw.shape[0]
    c = w_cls.shape[1]
    bf16 = jnp.bfloat16
    x = x_nchw[:, 0]
    xt = jnp.pad(x, ((0, 0), (1, 1), (1, 1))).transpose(1, 0, 2).astype(bf16)
    np_ = pl.cdiv(n, _B) * _B
    if np_ != n:
        xt = jnp.pad(xt, ((0, 0), (0, np_ - n), (0, 0)))
    a1c = a1.reshape(90, 896).astype(bf16)
    a2c = a2.astype(bf16)
    w1r = w_fc1.reshape(7, 448, 128).astype(bf16)
    out = pl.pallas_call(
        _net_kernel,
        out_shape=jax.ShapeDtypeStruct((np_, c), jnp.float32),
        grid_spec=pltpu.PrefetchScalarGridSpec(
            num_scalar_prefetch=0,
            grid=(np_ // _B,),
            in_specs=[
                pl.BlockSpec((30, _B, 30), lambda i: (0, i, 0)),
                pl.BlockSpec((90, 896), lambda i: (0, 0)),
                pl.BlockSpec((1, 896), lambda i: (0, 0)),
                pl.BlockSpec((3, 512, 896), lambda i: (0, 0, 0)),
                pl.BlockSpec((1, 896), lambda i: (0, 0)),
                pl.BlockSpec((7, 448, 128), lambda i: (0, 0, 0)),
                pl.BlockSpec((1, 128), lambda i: (0, 0)),
                pl.BlockSpec((128, c), lambda i: (0, 0)),
                pl.BlockSpec((1, c), lambda i: (0, 0)),
            ],
            out_specs=pl.BlockSpec((_B, c), lambda i: (i, 0)),
            scratch_shapes=[pltpu.VMEM((16, _B, 512), bf16)],
        ),
        compiler_params=pltpu.CompilerParams(
            dimension_semantics=("parallel",),
            vmem_limit_bytes=48 * 1024 * 1024,
        ),
    )(xt, a1c, b1, a2c, b2, w1r, b_fc1, w_cls, b_cls)
    return out[:n] if np_ != n else out
